# Initial kernel scaffold; baseline (speedup 1.0000x reference)
import jax
import jax.numpy as jnp
from jax.experimental import pallas as pl


def kernel(Z, ref_weight):
    raise NotImplementedError("write your pallas kernel here")


# speedup vs baseline: 463.6030x; 463.6030x over previous
import functools

import jax
import jax.numpy as jnp
from jax import lax
from jax.experimental import pallas as pl
from jax.experimental.pallas import tpu as pltpu
from jax.experimental.pallas import tpu_sc as plsc

NC = 2
NS = 16
NW = NC * NS
L = 16

UNROLL = 8
VEC = L * UNROLL


@functools.partial(jax.jit, static_argnames=("n_total", "chunk"))
def _sc_lookup_sum(z_flat, w_pad, n_total, chunk):
    per_worker = n_total // NW
    n_chunks = per_worker // chunk
    mesh = plsc.VectorSubcoreMesh(
        core_axis_name="c", subcore_axis_name="s", num_cores=NC, num_subcores=NS
    )

    @functools.partial(
        pl.kernel,
        out_type=jax.ShapeDtypeStruct((NW * L,), jnp.float32),
        mesh=mesh,
        compiler_params=pltpu.CompilerParams(needs_layout_passes=False),
        scratch_types=[
            pltpu.VMEM((128,), jnp.float32),
            pltpu.VMEM((chunk,), jnp.int32),
            pltpu.VMEM((chunk,), jnp.int32),
            pltpu.VMEM((L,), jnp.float32),
            pltpu.SemaphoreType.DMA,
            pltpu.SemaphoreType.DMA,
        ],
    )
    def k(z_hbm, w_hbm, out_hbm, tbl_v, buf0, buf1, acc_v, sem0, sem1):
        wid = lax.axis_index("s") * NC + lax.axis_index("c")
        base = wid * per_worker

        pltpu.sync_copy(w_hbm, tbl_v)

        bufs = (buf0, buf1)
        sems = (sem0, sem1)
        copies = [
            pltpu.async_copy(
                z_hbm.at[pl.ds(base + c * chunk, chunk)], bufs[c % 2], sems[c % 2]
            )
            for c in range(min(2, n_chunks))
        ]

        acc = jnp.zeros((L,), jnp.float32)
        for c in range(n_chunks):
            buf = bufs[c % 2]
            copies[c].wait()

            def body(i, a, buf=buf):
                off = i * VEC
                for j in range(UNROLL):
                    idx = buf[pl.ds(off + j * L, L)]
                    a = a + plsc.load_gather(tbl_v, [idx])
                return a

            acc = lax.fori_loop(0, chunk // VEC, body, acc)

            nxt = c + 2
            if nxt < n_chunks:
                copies.append(
                    pltpu.async_copy(
                        z_hbm.at[pl.ds(base + nxt * chunk, chunk)],
                        bufs[nxt % 2],
                        sems[nxt % 2],
                    )
                )

        acc_v[...] = acc
        pltpu.sync_copy(acc_v, out_hbm.at[pl.ds(wid * L, L)])

    return k(z_flat, w_pad)


def kernel(Z, ref_weight):
    n_total = Z.size
    z_flat = Z.reshape(n_total)
    w_pad = jnp.zeros((128,), jnp.float32).at[: ref_weight.shape[0]].set(
        ref_weight.reshape(-1)
    )
    partials = _sc_lookup_sum(z_flat, w_pad, n_total, 25600)
    return jnp.sum(partials)


# speedup vs baseline: 710.3091x; 1.5321x over previous
import functools

import jax
import jax.numpy as jnp
from jax import lax
from jax.experimental import pallas as pl
from jax.experimental.pallas import tpu as pltpu
from jax.experimental.pallas import tpu_sc as plsc

NC = 2
NS = 16
NW = NC * NS
L = 16


@functools.partial(jax.jit, static_argnames=("chunk_rows",))
def _sc_lookup_sum(z2d, w_pad, chunk_rows):
    n_rows, n_cols = z2d.shape
    rows_per_worker = n_rows // NW
    n_chunks = rows_per_worker // chunk_rows
    full_vecs = n_cols // L
    tail = n_cols - full_vecs * L
    mesh = plsc.VectorSubcoreMesh(
        core_axis_name="c", subcore_axis_name="s", num_cores=NC, num_subcores=NS
    )

    @functools.partial(
        pl.kernel,
        out_type=jax.ShapeDtypeStruct((NW * L,), jnp.float32),
        mesh=mesh,
        compiler_params=pltpu.CompilerParams(
            needs_layout_passes=False, use_tc_tiling_on_sc=True
        ),
        scratch_types=[
            pltpu.VMEM((128,), jnp.float32),
            pltpu.VMEM((chunk_rows, n_cols), jnp.int32),
            pltpu.VMEM((chunk_rows, n_cols), jnp.int32),
            pltpu.VMEM((L,), jnp.float32),
            pltpu.SemaphoreType.DMA,
            pltpu.SemaphoreType.DMA,
        ],
    )
    def k(z_hbm, w_hbm, out_hbm, tbl_v, buf0, buf1, acc_v, sem0, sem1):
        wid = lax.axis_index("s") * NC + lax.axis_index("c")
        base = wid * rows_per_worker

        pltpu.sync_copy(w_hbm, tbl_v)

        bufs = (buf0, buf1)
        sems = (sem0, sem1)
        copies = [
            pltpu.async_copy(
                z_hbm.at[pl.ds(base + c * chunk_rows, chunk_rows), :],
                bufs[c % 2],
                sems[c % 2],
            )
            for c in range(min(2, n_chunks))
        ]

        tail_keep = lax.iota(jnp.int32, L) >= (L - tail)
        zero = jnp.zeros((L,), jnp.float32)

        accs = (zero, zero, zero, zero)
        for c in range(n_chunks):
            buf = bufs[c % 2]
            copies[c].wait()

            def body(r, a, buf=buf):
                a = list(a)
                for j in range(full_vecs):
                    idx = buf[r, pl.ds(j * L, L)]
                    a[j % 4] = a[j % 4] + plsc.load_gather(tbl_v, [idx])
                idx = buf[r, pl.ds(n_cols - L, L)]
                v = plsc.load_gather(tbl_v, [idx])
                a[full_vecs % 4] = a[full_vecs % 4] + jnp.where(tail_keep, v, 0.0)
                return tuple(a)

            accs = lax.fori_loop(0, chunk_rows, body, accs)

            nxt = c + 2
            if nxt < n_chunks:
                copies.append(
                    pltpu.async_copy(
                        z_hbm.at[pl.ds(base + nxt * chunk_rows, chunk_rows), :],
                        bufs[nxt % 2],
                        sems[nxt % 2],
                    )
                )

        acc_v[...] = (accs[0] + accs[1]) + (accs[2] + accs[3])
        pltpu.sync_copy(acc_v, out_hbm.at[pl.ds(wid * L, L)])

    return k(z2d, w_pad)


def kernel(Z, ref_weight):
    w_pad = jnp.zeros((128,), jnp.float32).at[: ref_weight.shape[0]].set(
        ref_weight.reshape(-1)
    )
    partials = _sc_lookup_sum(Z, w_pad, 64)
    return jnp.sum(partials)


# speedup vs baseline: 965.8995x; 1.3598x over previous
import functools

import jax
import jax.numpy as jnp
from jax import lax
from jax.experimental import pallas as pl
from jax.experimental.pallas import tpu as pltpu
from jax.experimental.pallas import tpu_sc as plsc

NC = 2
NS = 16
NW = NC * NS
L = 16


@functools.partial(jax.jit, static_argnames=("chunk_rows",))
def _sc_lookup_sum(zt, w_pad, chunk_rows):
    n_rows, n_cols = zt.shape
    cols_per_worker = n_cols // NW
    n_chunks = n_rows // chunk_rows
    vecs_per_row = cols_per_worker // L
    mesh = plsc.VectorSubcoreMesh(
        core_axis_name="c", subcore_axis_name="s", num_cores=NC, num_subcores=NS
    )

    @functools.partial(
        pl.kernel,
        out_type=jax.ShapeDtypeStruct((NW * L,), jnp.float32),
        mesh=mesh,
        compiler_params=pltpu.CompilerParams(
            needs_layout_passes=False, use_tc_tiling_on_sc=True
        ),
        scratch_types=[
            pltpu.VMEM((128,), jnp.float32),
            pltpu.VMEM((chunk_rows, cols_per_worker), jnp.int32),
            pltpu.VMEM((chunk_rows, cols_per_worker), jnp.int32),
            pltpu.VMEM((L,), jnp.float32),
            pltpu.SemaphoreType.DMA,
            pltpu.SemaphoreType.DMA,
        ],
    )
    def k(z_hbm, w_hbm, out_hbm, tbl_v, buf0, buf1, acc_v, sem0, sem1):
        wid = lax.axis_index("s") * NC + lax.axis_index("c")
        col0 = wid * cols_per_worker

        pltpu.sync_copy(w_hbm, tbl_v)

        bufs = (buf0, buf1)
        sems = (sem0, sem1)
        copies = [
            pltpu.async_copy(
                z_hbm.at[
                    pl.ds(c * chunk_rows, chunk_rows),
                    pl.ds(col0, cols_per_worker),
                ],
                bufs[c % 2],
                sems[c % 2],
            )
            for c in range(min(2, n_chunks))
        ]

        zero = jnp.zeros((L,), jnp.float32)
        accs = (zero, zero, zero, zero)
        for c in range(n_chunks):
            buf = bufs[c % 2]
            copies[c].wait()

            def body(r, a, buf=buf):
                a = list(a)
                for j in range(vecs_per_row):
                    idx = buf[r, pl.ds(j * L, L)]
                    a[j % 4] = a[j % 4] + plsc.load_gather(tbl_v, [idx])
                return tuple(a)

            accs = lax.fori_loop(0, chunk_rows, body, accs)

            nxt = c + 2
            if nxt < n_chunks:
                copies.append(
                    pltpu.async_copy(
                        z_hbm.at[
                            pl.ds(nxt * chunk_rows, chunk_rows),
                            pl.ds(col0, cols_per_worker),
                        ],
                        bufs[nxt % 2],
                        sems[nxt % 2],
                    )
                )

        acc_v[...] = (accs[0] + accs[1]) + (accs[2] + accs[3])
        pltpu.sync_copy(acc_v, out_hbm.at[pl.ds(wid * L, L)])

    return k(zt, w_pad)


def kernel(Z, ref_weight):
    w_pad = jnp.zeros((128,), jnp.float32).at[: ref_weight.shape[0]].set(
        ref_weight.reshape(-1)
    )
    partials = _sc_lookup_sum(Z.T, w_pad, 40)
    return jnp.sum(partials)


# speedup vs baseline: 1008.2399x; 1.0438x over previous
import functools

import jax
import jax.numpy as jnp
from jax import lax
from jax.experimental import pallas as pl
from jax.experimental.pallas import tpu as pltpu
from jax.experimental.pallas import tpu_sc as plsc

NC = 2
NS = 16
NW = NC * NS
L = 16


@functools.partial(jax.jit, static_argnames=("chunk_rows",))
def _sc_lookup_sum(zt, w_pad, chunk_rows):
    n_rows, n_cols = zt.shape
    cols_per_worker = n_cols // NW
    n_chunks = n_rows // chunk_rows
    vecs_per_row = cols_per_worker // L
    mesh = plsc.VectorSubcoreMesh(
        core_axis_name="c", subcore_axis_name="s", num_cores=NC, num_subcores=NS
    )

    @functools.partial(
        pl.kernel,
        out_type=jax.ShapeDtypeStruct((NW * L,), jnp.float32),
        mesh=mesh,
        compiler_params=pltpu.CompilerParams(
            needs_layout_passes=False, use_tc_tiling_on_sc=True
        ),
        scratch_types=[
            pltpu.VMEM((128 * 128,), jnp.float32),
            pltpu.VMEM((chunk_rows, cols_per_worker), jnp.int32),
            pltpu.VMEM((chunk_rows, cols_per_worker), jnp.int32),
            pltpu.VMEM((L,), jnp.float32),
            pltpu.SemaphoreType.DMA,
            pltpu.SemaphoreType.DMA,
        ],
    )
    def k(z_hbm, w_hbm, out_hbm, tbl_v, buf0, buf1, acc_v, sem0, sem1):
        wid = lax.axis_index("s") * NC + lax.axis_index("c")
        col0 = wid * cols_per_worker

        bufs = (buf0, buf1)
        sems = (sem0, sem1)
        copies = [
            pltpu.async_copy(
                z_hbm.at[
                    pl.ds(c * chunk_rows, chunk_rows),
                    pl.ds(col0, cols_per_worker),
                ],
                bufs[c % 2],
                sems[c % 2],
            )
            for c in range(min(2, n_chunks))
        ]

        pltpu.sync_copy(w_hbm, tbl_v)

        zero = jnp.zeros((L,), jnp.float32)
        accs = (zero, zero, zero, zero)
        for c in range(n_chunks):
            buf = bufs[c % 2]
            copies[c].wait()

            def body(r, a, buf=buf):
                a = list(a)
                for j in range(vecs_per_row // 2):
                    ia = buf[r, pl.ds(2 * j * L, L)]
                    ib = buf[r, pl.ds((2 * j + 1) * L, L)]
                    idx = ia * 128 + ib
                    a[j % 4] = a[j % 4] + plsc.load_gather(tbl_v, [idx])
                return tuple(a)

            accs = lax.fori_loop(0, chunk_rows, body, accs)

            nxt = c + 2
            if nxt < n_chunks:
                copies.append(
                    pltpu.async_copy(
                        z_hbm.at[
                            pl.ds(nxt * chunk_rows, chunk_rows),
                            pl.ds(col0, cols_per_worker),
                        ],
                        bufs[nxt % 2],
                        sems[nxt % 2],
                    )
                )

        acc_v[...] = (accs[0] + accs[1]) + (accs[2] + accs[3])
        pltpu.sync_copy(acc_v, out_hbm.at[pl.ds(wid * L, L)])

    return k(zt, w_pad)


def kernel(Z, ref_weight):
    w_pad = jnp.zeros((128,), jnp.float32).at[: ref_weight.shape[0]].set(
        ref_weight.reshape(-1)
    )
    w_pair = (w_pad[:, None] + w_pad[None, :]).reshape(128 * 128)
    partials = _sc_lookup_sum(Z.T, w_pair, 40)
    return jnp.sum(partials)
